# baseline (speedup 1.0000x reference)
import functools

import jax
import jax.numpy as jnp
from jax import lax
from jax.experimental import pallas as pl
from jax.experimental.pallas import tpu as pltpu
from jax.experimental.pallas import tpu_sc as plsc

QN = 1024
D = 768
K = 100000
CH = 128
KP = 100352
NCH = KP // CH
NCHP = 896
KB = 1024
CPB = KB // CH
NKB = KP // KB
NSEL = 32
L = 200
LP = 640
MOFF = 256
SOFF = 512
BIG = 3.0e38
IBIG = 2**31 - 1

NC = 2
NS = 16
NW = NC * NS
QPT = QN // NW
SB = 8
NSB = QPT // SB


def _tc_body(q_ref, k_ref, q2_ref, k2_ref, d2_ref, sel_ref, tile_ref,
             cmin_ref):
    kstep = pl.program_id(0)
    qb = q_ref[...]
    kb = k_ref[...]
    dn = (((1,), (1,)), ((), ()))
    qk = lax.dot_general(qb.astype(jnp.bfloat16), kb.astype(jnp.bfloat16),
                         dn, preferred_element_type=jnp.float32)
    d2 = (q2_ref[...] + k2_ref[...]) - 2.0 * qk
    _store_chunks(d2, d2_ref, tile_ref, cmin_ref, kstep)

    @pl.when(kstep == NKB - 1)
    def _():
        _select_chunks(sel_ref, cmin_ref)


def _store_chunks(d2, d2_ref, tile_ref, cmin_ref, kstep):
    mins = []
    for j in range(CPB):
        sl = d2[:, j * CH:(j + 1) * CH]
        d2_ref[:, j, :] = sl
        mins.append(jnp.min(sl, axis=1, keepdims=True))
    m8 = jnp.concatenate(
        mins + [jnp.full((QN, 128 - CPB), BIG, jnp.float32)], axis=1)
    rolled = pltpu.roll(m8, (kstep % 16) * CPB, 1)
    tile = jnp.where(kstep % 16 == 0,
                     jnp.full((QN, 128), BIG, jnp.float32), tile_ref[...])
    tile_ref[...] = jnp.minimum(tile, rolled)

    @pl.when((kstep % 16 == 15) | (kstep == NKB - 1))
    def _():
        base = pl.multiple_of((kstep // 16) * 128, 128)
        cmin_ref[:, pl.ds(base, 128)] = tile_ref[...]


def _select_chunks(sel_ref, cmin_ref):
    lane = lax.broadcasted_iota(jnp.int32, (QN, NCHP), 1)
    lane32 = lax.broadcasted_iota(jnp.int32, (QN, NSEL), 1)

    def step(i, sel):
        c = cmin_ref[...]
        m = jnp.min(c, axis=1, keepdims=True)
        idx = jnp.min(jnp.where(c == m, lane, NCHP), axis=1, keepdims=True)
        sel = jnp.where(lane32 == i, idx, sel)
        cmin_ref[...] = jnp.where(lane == idx, BIG, c)
        return sel

    sel_ref[...] = lax.fori_loop(
        0, NSEL, step, jnp.zeros((QN, NSEL), jnp.int32))


def _tc_distance_select(queries, keys_p, q2, k2p):
    return pl.pallas_call(
        _tc_body,
        grid=(NKB,),
        in_specs=[
            pl.BlockSpec((QN, D), lambda k: (0, 0)),
            pl.BlockSpec((KB, D), lambda k: (k, 0)),
            pl.BlockSpec((QN, 1), lambda k: (0, 0)),
            pl.BlockSpec((1, KB), lambda k: (0, k)),
        ],
        out_specs=[
            pl.BlockSpec((QN, CPB, CH), lambda k: (0, k, 0)),
            pl.BlockSpec((QN, NSEL), lambda k: (0, 0)),
        ],
        out_shape=[
            jax.ShapeDtypeStruct((QN, NCH, CH), jnp.float32),
            jax.ShapeDtypeStruct((QN, NSEL), jnp.int32),
        ],
        scratch_shapes=[
            pltpu.VMEM((QN, 128), jnp.float32),
            pltpu.VMEM((QN, NCHP), jnp.float32),
        ],
        compiler_params=pltpu.CompilerParams(
            dimension_semantics=("arbitrary",),
        ),
    )(queries, keys_p, q2, k2p)


def _iota16():
    return lax.iota(jnp.int32, 16)


def _sc_body(d2f, selh, payh,
             otdh, oindh, opayh, osth, oenh,
             cid_v, idx_v, gath_v, cand_v, candg_v,
             otd_v, oind_v, ost_v, oen_v, pidx_v, pay_v, sem):
    wid = lax.axis_index("s") * NC + lax.axis_index("c")
    qbase = wid * QPT

    pltpu.sync_copy(selh.at[pl.ds(qbase, QPT)], cid_v)

    def sub_batch(sb, _):
        q0 = qbase + sb * SB

        def mk_idx(qi, _):
            for h in range(2):
                cid = cid_v[sb * SB + qi, pl.ds(h * 16, 16)]
                idx_v[pl.ds(qi * NSEL + h * 16, 16)] = (
                    (q0 + qi) * NCH + cid)
            return 0
        lax.fori_loop(0, SB, mk_idx, 0)

        pltpu.async_copy(d2f.at[idx_v], gath_v, sem).wait()

        def select_q(qi, _):
            def chunk_min(c, t0):
                m = gath_v[qi * NSEL + c, pl.ds(0, 16)]
                for o in range(1, 8):
                    m = jnp.minimum(m, gath_v[qi * NSEL + c, pl.ds(o * 16, 16)])
                return jnp.maximum(t0, jnp.min(m))
            t0 = lax.fori_loop(0, NSEL, chunk_min, -BIG)

            def compact(c, cursor):
                cidv = plsc.load_gather(
                    cid_v, [jnp.full((16,), sb * SB + qi, jnp.int32),
                            jnp.full((16,), c, jnp.int32)])
                gbase = cidv * CH
                for o in range(8):
                    v = gath_v[qi * NSEL + c, pl.ds(o * 16, 16)]
                    g = gbase + (o * 16 + _iota16())
                    msk = v <= t0
                    plsc.store_compressed(cand_v.at[pl.ds(cursor, 16)], v,
                                          mask=msk)
                    plsc.store_compressed(candg_v.at[pl.ds(cursor, 16)], g,
                                          mask=msk)
                    cnt = plsc.all_reduce_population_count(msk)
                    cnt = jnp.max(cnt) if cnt.ndim else cnt
                    cursor = cursor + cnt
                return cursor
            cursor = lax.fori_loop(0, NSEL, compact, jnp.int32(0))

            cand_v[pl.ds(cursor, 16)] = jnp.full((16,), BIG, jnp.float32)
            candg_v[pl.ds(cursor, 16)] = jnp.full((16,), IBIG, jnp.int32)
            nv = (cursor + 15) // 16

            def extract(i, carry):
                od0, od1, oi0, oi1 = carry

                def vmin_pass(j, acc):
                    return jnp.minimum(acc, cand_v[pl.ds(j * 16, 16)])
                vmin = jnp.min(lax.fori_loop(
                    0, nv, vmin_pass, jnp.full((16,), BIG, jnp.float32)))

                def gmin_pass(j, acc):
                    v = cand_v[pl.ds(j * 16, 16)]
                    g = candg_v[pl.ds(j * 16, 16)]
                    return jnp.minimum(acc, jnp.where(v == vmin, g, IBIG))
                gmin = jnp.min(lax.fori_loop(
                    0, nv, gmin_pass, jnp.full((16,), IBIG, jnp.int32)))

                def kill_pass(j, _):
                    v = cand_v[pl.ds(j * 16, 16)]
                    g = candg_v[pl.ds(j * 16, 16)]
                    cand_v[pl.ds(j * 16, 16)] = jnp.where(
                        (v == vmin) & (g == gmin), BIG, v)
                    return 0
                lax.fori_loop(0, nv, kill_pass, 0)

                lm = _iota16() == (i % 16)
                lo = i < 16
                od0 = jnp.where(lm & lo, vmin, od0)
                oi0 = jnp.where(lm & lo, gmin, oi0)
                od1 = jnp.where(lm & (~lo), vmin, od1)
                oi1 = jnp.where(lm & (~lo), gmin, oi1)
                return od0, od1, oi0, oi1

            z16f = jnp.zeros((16,), jnp.float32)
            z16i = jnp.zeros((16,), jnp.int32)
            od0, od1, oi0, oi1 = lax.fori_loop(
                0, NSEL, extract, (z16f, z16f, z16i, z16i))

            otd_v[qi, pl.ds(0, 16)] = od0
            otd_v[qi, pl.ds(16, 16)] = od1
            oind_v[qi, pl.ds(0, 16)] = oi0
            oind_v[qi, pl.ds(16, 16)] = oi1
            pidx_v[qi // 2, pl.ds((qi % 2) * NSEL, 16)] = oi0
            pidx_v[qi // 2, pl.ds((qi % 2) * NSEL + 16, 16)] = oi1
            return 0
        lax.fori_loop(0, SB, select_q, 0)

        pltpu.sync_copy(otd_v, otdh.at[pl.ds(q0, SB)])
        pltpu.sync_copy(oind_v, oindh.at[pl.ds(q0, SB)])

        rows0 = q0 * NSEL
        qtr = SB * NSEL // 4
        for hh in range(4):
            pltpu.async_copy(payh.at[pidx_v.at[hh]], pay_v, sem).wait()
            pltpu.sync_copy(pay_v, opayh.at[pl.ds(rows0 + hh * qtr, qtr)])
            for ql in range(2):
                qi = hh * 2 + ql
                for h in range(2):
                    rows = ql * NSEL + h * 16 + _iota16()
                    ost_v[qi, pl.ds(h * 16, 16)] = plsc.load_gather(
                        pay_v, [rows, jnp.full((16,), SOFF, jnp.int32)])
                    oen_v[qi, pl.ds(h * 16, 16)] = plsc.load_gather(
                        pay_v, [rows, jnp.full((16,), SOFF + 1, jnp.int32)])
        pltpu.sync_copy(ost_v, osth.at[pl.ds(q0, SB)])
        pltpu.sync_copy(oen_v, oenh.at[pl.ds(q0, SB)])
        return 0

    lax.fori_loop(0, NSB, sub_batch, 0)


def _sc_topk_gather(d2f, sel_ids, pay):
    mesh = plsc.VectorSubcoreMesh(core_axis_name="c", subcore_axis_name="s",
                                  num_cores=NC, num_subcores=NS)
    fn = functools.partial(
        pl.kernel,
        out_type=(
            jax.ShapeDtypeStruct((QN, NSEL), jnp.float32),
            jax.ShapeDtypeStruct((QN, NSEL), jnp.int32),
            jax.ShapeDtypeStruct((QN * NSEL, LP), jnp.int32),
            jax.ShapeDtypeStruct((QN, NSEL), jnp.int32),
            jax.ShapeDtypeStruct((QN, NSEL), jnp.int32),
        ),
        mesh=mesh,
        scratch_types=[
            pltpu.VMEM((QPT, NSEL), jnp.int32),
            pltpu.VMEM((SB * NSEL,), jnp.int32),
            pltpu.VMEM((SB * NSEL, CH), jnp.float32),
            pltpu.VMEM((NSEL * CH + 16,), jnp.float32),
            pltpu.VMEM((NSEL * CH + 16,), jnp.int32),
            pltpu.VMEM((SB, NSEL), jnp.float32),
            pltpu.VMEM((SB, NSEL), jnp.int32),
            pltpu.VMEM((SB, NSEL), jnp.int32),
            pltpu.VMEM((SB, NSEL), jnp.int32),
            pltpu.VMEM((4, SB * NSEL // 4), jnp.int32),
            pltpu.VMEM((SB * NSEL // 4, LP), jnp.int32),
            pltpu.SemaphoreType.DMA,
        ],
        compiler_params=pltpu.CompilerParams(needs_layout_passes=False),
    )(_sc_body)
    return fn(d2f, sel_ids, pay)


def _slice_body(p_ref, ids_ref, mks_ref):
    ids_ref[...] = p_ref[:, :, :L]
    mks_ref[...] = p_ref[:, :, MOFF:MOFF + L]


def _tc_slice(opay3):
    return pl.pallas_call(
        _slice_body,
        grid=(16,),
        in_specs=[pl.BlockSpec((QN // 16, NSEL, LP), lambda i: (i, 0, 0))],
        out_specs=[
            pl.BlockSpec((QN // 16, NSEL, L), lambda i: (i, 0, 0)),
            pl.BlockSpec((QN // 16, NSEL, L), lambda i: (i, 0, 0)),
        ],
        out_shape=[
            jax.ShapeDtypeStruct((QN, NSEL, L), jnp.int32),
            jax.ShapeDtypeStruct((QN, NSEL, L), jnp.int32),
        ],
    )(opay3)


def kernel(queries, keys, stored_input_ids, stored_masks,
           start_positions, end_positions):
    keys_p = jnp.pad(keys, ((0, KP - K), (0, 0)))
    q2 = jnp.sum(queries * queries, axis=1, keepdims=True)
    k2 = jnp.sum(keys * keys, axis=1)
    k2p = jnp.concatenate([k2, jnp.full((KP - K,), BIG, jnp.float32)])[None, :]
    pay = jnp.concatenate(
        [stored_input_ids, jnp.zeros((K, MOFF - L), jnp.int32),
         stored_masks, jnp.zeros((K, SOFF - MOFF - L), jnp.int32),
         start_positions[:, None], end_positions[:, None],
         jnp.zeros((K, LP - SOFF - 2), jnp.int32)], axis=1)
    d2r, sel_ids = _tc_distance_select(queries, keys_p, q2, k2p)
    d2f = d2r.reshape(QN * NCH, CH)
    otd, oind, opay, ost, oen = _sc_topk_gather(d2f, sel_ids, pay)
    oids, omks = _tc_slice(opay.reshape(QN, NSEL, LP))
    return (otd, oids, omks, ost, oen, oind)


# speedup vs baseline: 1.0668x; 1.0668x over previous
import functools

import jax
import jax.numpy as jnp
from jax import lax
from jax.experimental import pallas as pl
from jax.experimental.pallas import tpu as pltpu
from jax.experimental.pallas import tpu_sc as plsc

QN = 1024
D = 768
K = 100000
CH = 128
KP = 100352
NCH = KP // CH
NCHP = 896
KB = 1024
CPB = KB // CH
NKB = KP // KB
NSEL = 32
L = 200
LP = 512
BIG = 3.0e38
IBIG = 2**31 - 1

NC = 2
NS = 16
NW = NC * NS
QPT = QN // NW
SB = 8
NSB = QPT // SB


def _tc_body(q_ref, k_ref, q2_ref, k2_ref, d2_ref, sel_ref, tile_ref,
             cmin_ref):
    kstep = pl.program_id(0)
    qb = q_ref[...]
    kb = k_ref[...]
    dn = (((1,), (1,)), ((), ()))
    qk = lax.dot_general(qb.astype(jnp.bfloat16), kb.astype(jnp.bfloat16),
                         dn, preferred_element_type=jnp.float32)
    d2 = (q2_ref[...] + k2_ref[...]) - 2.0 * qk
    _store_chunks(d2, d2_ref, tile_ref, cmin_ref, kstep)

    @pl.when(kstep == NKB - 1)
    def _():
        _select_chunks(sel_ref, cmin_ref)


def _store_chunks(d2, d2_ref, tile_ref, cmin_ref, kstep):
    mins = []
    for j in range(CPB):
        sl = d2[:, j * CH:(j + 1) * CH]
        d2_ref[:, j, :] = sl
        mins.append(jnp.min(sl, axis=1, keepdims=True))
    m8 = jnp.concatenate(
        mins + [jnp.full((QN, 128 - CPB), BIG, jnp.float32)], axis=1)
    rolled = pltpu.roll(m8, (kstep % 16) * CPB, 1)
    tile = jnp.where(kstep % 16 == 0,
                     jnp.full((QN, 128), BIG, jnp.float32), tile_ref[...])
    tile_ref[...] = jnp.minimum(tile, rolled)

    @pl.when((kstep % 16 == 15) | (kstep == NKB - 1))
    def _():
        base = pl.multiple_of((kstep // 16) * 128, 128)
        cmin_ref[:, pl.ds(base, 128)] = tile_ref[...]


def _select_chunks(sel_ref, cmin_ref):
    lane = lax.broadcasted_iota(jnp.int32, (QN, NCHP), 1)
    lane32 = lax.broadcasted_iota(jnp.int32, (QN, NSEL), 1)

    def step(i, sel):
        c = cmin_ref[...]
        m = jnp.min(c, axis=1, keepdims=True)
        idx = jnp.min(jnp.where(c == m, lane, NCHP), axis=1, keepdims=True)
        sel = jnp.where(lane32 == i, idx, sel)
        cmin_ref[...] = jnp.where(lane == idx, BIG, c)
        return sel

    sel_ref[...] = lax.fori_loop(
        0, NSEL, step, jnp.zeros((QN, NSEL), jnp.int32))


def _tc_distance_select(queries, keys_p, q2, k2p):
    return pl.pallas_call(
        _tc_body,
        grid=(NKB,),
        in_specs=[
            pl.BlockSpec((QN, D), lambda k: (0, 0)),
            pl.BlockSpec((KB, D), lambda k: (k, 0)),
            pl.BlockSpec((QN, 1), lambda k: (0, 0)),
            pl.BlockSpec((1, KB), lambda k: (0, k)),
        ],
        out_specs=[
            pl.BlockSpec((QN, CPB, CH), lambda k: (0, k, 0)),
            pl.BlockSpec((QN, NSEL), lambda k: (0, 0)),
        ],
        out_shape=[
            jax.ShapeDtypeStruct((QN, NCH, CH), jnp.float32),
            jax.ShapeDtypeStruct((QN, NSEL), jnp.int32),
        ],
        scratch_shapes=[
            pltpu.VMEM((QN, 128), jnp.float32),
            pltpu.VMEM((QN, NCHP), jnp.float32),
        ],
        compiler_params=pltpu.CompilerParams(
            dimension_semantics=("arbitrary",),
        ),
    )(queries, keys_p, q2, k2p)


def _iota16():
    return lax.iota(jnp.int32, 16)


def _sc_body(d2f, selh, payh,
             otdh, oindh, opayh,
             cid_v, idx_v, gath_v, cand_v, candg_v,
             otd_v, oind_v, pidx_v, pay_v, sem):
    wid = lax.axis_index("s") * NC + lax.axis_index("c")
    qbase = wid * QPT

    pltpu.sync_copy(selh.at[pl.ds(qbase, QPT)], cid_v)

    def sub_batch(sb, _):
        q0 = qbase + sb * SB

        def mk_idx(qi, _):
            for h in range(2):
                cid = cid_v[sb * SB + qi, pl.ds(h * 16, 16)]
                idx_v[pl.ds(qi * NSEL + h * 16, 16)] = (
                    (q0 + qi) * NCH + cid)
            return 0
        lax.fori_loop(0, SB, mk_idx, 0)

        pltpu.async_copy(d2f.at[idx_v], gath_v, sem).wait()

        def select_q(qi, _):
            def chunk_min(c, t0):
                m = gath_v[qi * NSEL + c, pl.ds(0, 16)]
                for o in range(1, 8):
                    m = jnp.minimum(m, gath_v[qi * NSEL + c, pl.ds(o * 16, 16)])
                return jnp.maximum(t0, jnp.min(m))
            t0 = lax.fori_loop(0, NSEL, chunk_min, -BIG)

            def compact(c, cursor):
                cidv = plsc.load_gather(
                    cid_v, [jnp.full((16,), sb * SB + qi, jnp.int32),
                            jnp.full((16,), c, jnp.int32)])
                gbase = cidv * CH
                for o in range(8):
                    v = gath_v[qi * NSEL + c, pl.ds(o * 16, 16)]
                    g = gbase + (o * 16 + _iota16())
                    msk = v <= t0
                    plsc.store_compressed(cand_v.at[pl.ds(cursor, 16)], v,
                                          mask=msk)
                    plsc.store_compressed(candg_v.at[pl.ds(cursor, 16)], g,
                                          mask=msk)
                    cnt = plsc.all_reduce_population_count(msk)
                    cnt = jnp.max(cnt) if cnt.ndim else cnt
                    cursor = cursor + cnt
                return cursor
            cursor = lax.fori_loop(0, NSEL, compact, jnp.int32(0))

            cand_v[pl.ds(cursor, 16)] = jnp.full((16,), BIG, jnp.float32)
            candg_v[pl.ds(cursor, 16)] = jnp.full((16,), IBIG, jnp.int32)
            nv = (cursor + 15) // 16

            def extract(i, carry):
                od0, od1, oi0, oi1 = carry

                def vmin_pass(j, acc):
                    return jnp.minimum(acc, cand_v[pl.ds(j * 16, 16)])
                vmin = jnp.min(lax.fori_loop(
                    0, nv, vmin_pass, jnp.full((16,), BIG, jnp.float32)))

                def gmin_pass(j, acc):
                    v = cand_v[pl.ds(j * 16, 16)]
                    g = candg_v[pl.ds(j * 16, 16)]
                    return jnp.minimum(acc, jnp.where(v == vmin, g, IBIG))
                gmin = jnp.min(lax.fori_loop(
                    0, nv, gmin_pass, jnp.full((16,), IBIG, jnp.int32)))

                def kill_pass(j, _):
                    v = cand_v[pl.ds(j * 16, 16)]
                    g = candg_v[pl.ds(j * 16, 16)]
                    cand_v[pl.ds(j * 16, 16)] = jnp.where(
                        (v == vmin) & (g == gmin), BIG, v)
                    return 0
                lax.fori_loop(0, nv, kill_pass, 0)

                lm = _iota16() == (i % 16)
                lo = i < 16
                od0 = jnp.where(lm & lo, vmin, od0)
                oi0 = jnp.where(lm & lo, gmin, oi0)
                od1 = jnp.where(lm & (~lo), vmin, od1)
                oi1 = jnp.where(lm & (~lo), gmin, oi1)
                return od0, od1, oi0, oi1

            z16f = jnp.zeros((16,), jnp.float32)
            z16i = jnp.zeros((16,), jnp.int32)
            od0, od1, oi0, oi1 = lax.fori_loop(
                0, NSEL, extract, (z16f, z16f, z16i, z16i))

            otd_v[qi, pl.ds(0, 16)] = od0
            otd_v[qi, pl.ds(16, 16)] = od1
            oind_v[qi, pl.ds(0, 16)] = oi0
            oind_v[qi, pl.ds(16, 16)] = oi1
            pidx_v[qi // 4, pl.ds((qi % 4) * NSEL, 16)] = oi0
            pidx_v[qi // 4, pl.ds((qi % 4) * NSEL + 16, 16)] = oi1
            return 0
        lax.fori_loop(0, SB, select_q, 0)

        pltpu.sync_copy(otd_v, otdh.at[pl.ds(q0, SB)])
        pltpu.sync_copy(oind_v, oindh.at[pl.ds(q0, SB)])

        rows0 = q0 * NSEL
        half = SB * NSEL // 2
        for hh in range(2):
            pltpu.async_copy(payh.at[pidx_v.at[hh]], pay_v, sem).wait()
            pltpu.sync_copy(pay_v, opayh.at[pl.ds(rows0 + hh * half, half)])
        return 0

    lax.fori_loop(0, NSB, sub_batch, 0)


def _sc_topk_gather(d2f, sel_ids, pay):
    mesh = plsc.VectorSubcoreMesh(core_axis_name="c", subcore_axis_name="s",
                                  num_cores=NC, num_subcores=NS)
    fn = functools.partial(
        pl.kernel,
        out_type=(
            jax.ShapeDtypeStruct((QN, NSEL), jnp.float32),
            jax.ShapeDtypeStruct((QN, NSEL), jnp.int32),
            jax.ShapeDtypeStruct((QN * NSEL, LP), jnp.int32),
        ),
        mesh=mesh,
        scratch_types=[
            pltpu.VMEM((QPT, NSEL), jnp.int32),
            pltpu.VMEM((SB * NSEL,), jnp.int32),
            pltpu.VMEM((SB * NSEL, CH), jnp.float32),
            pltpu.VMEM((NSEL * CH + 16,), jnp.float32),
            pltpu.VMEM((NSEL * CH + 16,), jnp.int32),
            pltpu.VMEM((SB, NSEL), jnp.float32),
            pltpu.VMEM((SB, NSEL), jnp.int32),
            pltpu.VMEM((2, SB * NSEL // 2), jnp.int32),
            pltpu.VMEM((SB * NSEL // 2, LP), jnp.int32),
            pltpu.SemaphoreType.DMA,
        ],
        compiler_params=pltpu.CompilerParams(needs_layout_passes=False),
    )(_sc_body)
    return fn(d2f, sel_ids, pay)


def kernel(queries, keys, stored_input_ids, stored_masks,
           start_positions, end_positions):
    keys_p = jnp.pad(keys, ((0, KP - K), (0, 0)))
    q2 = jnp.sum(queries * queries, axis=1, keepdims=True)
    k2 = jnp.sum(keys * keys, axis=1)
    k2p = jnp.concatenate([k2, jnp.full((KP - K,), BIG, jnp.float32)])[None, :]
    pay = jnp.concatenate(
        [stored_input_ids, stored_masks,
         start_positions[:, None], end_positions[:, None],
         jnp.zeros((K, LP - 2 * L - 2), jnp.int32)], axis=1)
    d2r, sel_ids = _tc_distance_select(queries, keys_p, q2, k2p)
    d2f = d2r.reshape(QN * NCH, CH)
    otd, oind, opay = _sc_topk_gather(d2f, sel_ids, pay)
    return (otd,
            opay[:, :L].reshape(QN, NSEL, L),
            opay[:, L:2 * L].reshape(QN, NSEL, L),
            opay[:, 2 * L].reshape(QN, NSEL),
            opay[:, 2 * L + 1].reshape(QN, NSEL),
            oind)
